# Initial kernel scaffold; baseline (speedup 1.0000x reference)
import jax
import jax.numpy as jnp
from jax.experimental import pallas as pl


def kernel(features, labels, text_embeddings, ignore_index):
    raise NotImplementedError("write your pallas kernel here")


# speedup vs baseline: 2.1741x; 2.1741x over previous
import functools

import jax
import jax.numpy as jnp
from jax import lax
from jax.experimental import pallas as pl
from jax.experimental.pallas import tpu as pltpu
from jax.experimental.pallas import tpu_sc as plsc

TEMP = 0.1
NUM_CLASSES = 150
CPAD = 160
NC, NS, L = 2, 16, 16
NW = NC * NS
CH = 96
BATCH = 4
HW = 512 * 512
PIX = BATCH * HW
P = 8192
CH_PER_TILE = CH // NW
CHUNKS = HW // P
CNT_PER_TILE = PIX // NW
CNT_CHUNKS = CNT_PER_TILE // P


def _sc_segsum_body(feat_hbm, lab_hbm, ign_hbm, sums_hbm, cnts_hbm,
                    lab_v, feat_v0, feat_v1, feat_v2, acc_v, cnt_v, ign_v):
    feat_v = (feat_v0, feat_v1, feat_v2)
    cid = lax.axis_index("c")
    sid = lax.axis_index("s")
    wid = sid * NC + cid
    ch0 = wid * CH_PER_TILE

    pltpu.sync_copy(ign_hbm, ign_v)
    ign = ign_v[...]

    zf = jnp.zeros((L,), jnp.float32)

    def zacc(i, c):
        acc_v[pl.ds(i * L, L)] = zf
        return c
    lax.fori_loop(0, (CH_PER_TILE * CPAD) // L, zacc, 0)

    def zcnt(i, c):
        cnt_v[pl.ds(i * L, L)] = zf
        return c
    lax.fori_loop(0, CPAD // L, zcnt, 0)

    for b in range(BATCH):
        def chunk_body(k, c):
            pltpu.sync_copy(lab_hbm.at[pl.ds(b * HW + k * P, P)], lab_v)
            for ci in range(CH_PER_TILE):
                plane = b * CH + ch0 + ci
                pltpu.sync_copy(feat_hbm.at[plane, pl.ds(k * P, P)],
                                feat_v[ci])

            def pix_body(i, cc):
                lv = lab_v[pl.ds(i * L, L)]
                lv = jnp.where(lv == ign, 0, lv)
                for ci in range(CH_PER_TILE):
                    fv = feat_v[ci][pl.ds(i * L, L)]
                    plsc.addupdate_scatter(acc_v, [lv + (ci * CPAD)], fv)
                return cc
            lax.fori_loop(0, P // L, pix_body, 0)
            return c
        lax.fori_loop(0, CHUNKS, chunk_body, 0)

    ones = jnp.ones((L,), jnp.float32)
    base = wid * CNT_PER_TILE

    def cnt_chunk(k, c):
        pltpu.sync_copy(lab_hbm.at[pl.ds(base + k * P, P)], lab_v)

        def cnt_body(i, cc):
            lv = lab_v[pl.ds(i * L, L)]
            lv = jnp.where(lv == ign, 0, lv)
            plsc.addupdate_scatter(cnt_v, [lv], ones)
            return cc
        lax.fori_loop(0, P // L, cnt_body, 0)
        return c
    lax.fori_loop(0, CNT_CHUNKS, cnt_chunk, 0)

    for ci in range(CH_PER_TILE):
        pltpu.sync_copy(acc_v.at[pl.ds(ci * CPAD, CPAD)],
                        sums_hbm.at[ch0 + ci])
    pltpu.sync_copy(cnt_v, cnts_hbm.at[wid])


_sc_segsum = functools.partial(
    pl.kernel,
    out_type=(jax.ShapeDtypeStruct((CH, CPAD), jnp.float32),
              jax.ShapeDtypeStruct((NW, CPAD), jnp.float32)),
    mesh=plsc.VectorSubcoreMesh(core_axis_name="c", subcore_axis_name="s"),
    compiler_params=pltpu.CompilerParams(needs_layout_passes=False,
                                         use_tc_tiling_on_sc=False),
    scratch_types=[
        pltpu.VMEM((P,), jnp.int32),
        pltpu.VMEM((P,), jnp.float32),
        pltpu.VMEM((P,), jnp.float32),
        pltpu.VMEM((P,), jnp.float32),
        pltpu.VMEM((CH_PER_TILE * CPAD,), jnp.float32),
        pltpu.VMEM((CPAD,), jnp.float32),
        pltpu.VMEM((L,), jnp.int32),
    ],
)(_sc_segsum_body)


def _loss_body(sums_ref, cnts_ref, cnts_t_ref, text_ref, out_ref):
    counts_row = jnp.sum(cnts_ref[...], axis=0, keepdims=True)
    counts_col = jnp.sum(cnts_t_ref[...], axis=1, keepdims=True)

    s = sums_ref[...]
    protos = jnp.where(counts_row > 0.0,
                       s / jnp.maximum(counts_row, 1.0), 0.0)
    pnorm = jnp.sqrt(jnp.sum(protos * protos, axis=0, keepdims=True))
    protos = protos / jnp.maximum(pnorm, 1e-12)

    t = text_ref[...]
    tnorm = jnp.sqrt(jnp.sum(t * t, axis=1, keepdims=True))
    tn = t / jnp.maximum(tnorm, 1e-12)

    sim_t = lax.dot_general(tn, protos, (((1,), (0,)), ((), ())),
                            preferred_element_type=jnp.float32) / TEMP

    valid_j = counts_col[:NUM_CLASSES, :] > 0.0
    neg_inf = jnp.float32(-jnp.inf)
    masked = jnp.where(valid_j, sim_t, neg_inf)

    m = jnp.max(masked, axis=0, keepdims=True)
    z = jnp.exp(masked - m)
    lse = jnp.log(jnp.sum(z, axis=0, keepdims=True)) + m
    logp = masked - lse

    jj = lax.broadcasted_iota(jnp.int32, (NUM_CLASSES, CPAD), 0)
    ii = lax.broadcasted_iota(jnp.int32, (NUM_CLASSES, CPAD), 1)
    eye = (jj == ii).astype(jnp.float32)
    contrib = jnp.where(valid_j, eye * logp, 0.0)
    loss_sum = -jnp.sum(contrib, axis=(0, 1), keepdims=True)

    n_valid = jnp.sum(valid_j.astype(jnp.float32), axis=(0, 1), keepdims=True)
    out_ref[...] = jnp.where(n_valid > 1.0, loss_sum / n_valid, 0.0)


def kernel(features, labels, text_embeddings, ignore_index):
    feat2d = features.reshape(BATCH * CH, HW)
    lab1d = labels.reshape(PIX).astype(jnp.int32)
    ign = jnp.full((L,), ignore_index, dtype=jnp.int32)

    sums, cnts = _sc_segsum(feat2d, lab1d, ign)

    loss2d = pl.pallas_call(
        _loss_body,
        out_shape=jax.ShapeDtypeStruct((1, 1), jnp.float32),
    )(sums, cnts, cnts.T, text_embeddings.astype(jnp.float32))
    return loss2d[0, 0]


# speedup vs baseline: 3.1951x; 1.4696x over previous
import functools

import jax
import jax.numpy as jnp
from jax import lax
from jax.experimental import pallas as pl
from jax.experimental.pallas import tpu as pltpu
from jax.experimental.pallas import tpu_sc as plsc

TEMP = 0.1
NUM_CLASSES = 150
CPAD = 160
NC, NS, L = 2, 16, 16
NW = NC * NS
CH = 96
BATCH = 4
HW = 512 * 512
PIX = BATCH * HW
SLICE = PIX // NW
P = 4096
G = 8
CHUNKS = SLICE // P
BLOCKS = CH // G
STEPS = CHUNKS * BLOCKS
ACC_N = CH * CPAD


def _sc_segsum_body(feat_hbm, lab_hbm, ign_hbm, sums_hbm, cnts_hbm,
                    lab_v, f0, f1, acc_v, cnt_v, ign_v, sem0, sem1):
    fbuf = (f0, f1)
    sem = (sem0, sem1)
    cid = lax.axis_index("c")
    sid = lax.axis_index("s")
    wid = sid * NC + cid
    b = wid // 8
    hw0 = (wid % 8) * SLICE

    pltpu.sync_copy(ign_hbm, ign_v)
    ign = ign_v[...]

    pltpu.sync_copy(lab_hbm.at[pl.ds(wid * SLICE, SLICE)], lab_v)

    zf = jnp.zeros((L,), jnp.float32)

    def zacc(i, c):
        acc_v[pl.ds(i * L, L)] = zf
        return c
    lax.fori_loop(0, ACC_N // L, zacc, 0)

    def zcnt(i, c):
        cnt_v[pl.ds(i * L, L)] = zf
        return c
    lax.fori_loop(0, CPAD // L, zcnt, 0)

    def _copies(step, par):
        k = step // BLOCKS
        g = step % BLOCKS
        return [pltpu.make_async_copy(
            feat_hbm.at[b * CH + g * G + ci, pl.ds(hw0 + k * P, P)],
            fbuf[par][ci], sem[par]) for ci in range(G)]

    def issue(step, par):
        for c in _copies(step, par):
            c.start()

    def wait(step, par):
        for c in _copies(step, par):
            c.wait()

    issue(0, 0)
    issue(1, 1)

    def compute(step, par):
        k = step // BLOCKS
        g = step % BLOCKS

        def pix(i, c):
            lv = lab_v[pl.ds(k * P + i * L, L)]
            lv = jnp.where(lv == ign, 0, lv)
            lvg = lv + g * (G * CPAD)
            for ci in range(G):
                fv = fbuf[par][ci][pl.ds(i * L, L)]
                plsc.addupdate_scatter(acc_v, [lvg + ci * CPAD], fv)
            return c
        lax.fori_loop(0, P // L, pix, 0)

    def loop(kk, c):
        for par in range(2):
            step = kk * 2 + par
            wait(step, par)
            compute(step, par)

            @pl.when(step + 2 < STEPS)
            def _():
                issue(step + 2, par)
        return c
    lax.fori_loop(0, STEPS // 2, loop, 0)

    ones = jnp.ones((L,), jnp.float32)

    def cnt_body(i, c):
        lv = lab_v[pl.ds(i * L, L)]
        lv = jnp.where(lv == ign, 0, lv)
        plsc.addupdate_scatter(cnt_v, [lv], ones)
        return c
    lax.fori_loop(0, SLICE // L, cnt_body, 0)

    pltpu.sync_copy(acc_v, sums_hbm.at[wid])
    pltpu.sync_copy(cnt_v, cnts_hbm.at[wid])


_sc_segsum = functools.partial(
    pl.kernel,
    out_type=(jax.ShapeDtypeStruct((NW, ACC_N), jnp.float32),
              jax.ShapeDtypeStruct((NW, CPAD), jnp.float32)),
    mesh=plsc.VectorSubcoreMesh(core_axis_name="c", subcore_axis_name="s"),
    compiler_params=pltpu.CompilerParams(needs_layout_passes=False,
                                         use_tc_tiling_on_sc=False),
    scratch_types=[
        pltpu.VMEM((SLICE,), jnp.int32),
        [pltpu.VMEM((P,), jnp.float32)] * G,
        [pltpu.VMEM((P,), jnp.float32)] * G,
        pltpu.VMEM((ACC_N,), jnp.float32),
        pltpu.VMEM((CPAD,), jnp.float32),
        pltpu.VMEM((L,), jnp.int32),
        pltpu.SemaphoreType.DMA,
        pltpu.SemaphoreType.DMA,
    ],
)(_sc_segsum_body)


def _loss_body(sums_ref, cnts_ref, cnts_t_ref, text_ref, out_ref):
    counts_row = jnp.sum(cnts_ref[...], axis=0, keepdims=True)
    counts_col = jnp.sum(cnts_t_ref[...], axis=1, keepdims=True)

    s3 = sums_ref[...]
    s = jnp.sum(s3, axis=0)
    protos = jnp.where(counts_row > 0.0,
                       s / jnp.maximum(counts_row, 1.0), 0.0)
    pnorm = jnp.sqrt(jnp.sum(protos * protos, axis=0, keepdims=True))
    protos = protos / jnp.maximum(pnorm, 1e-12)

    t = text_ref[...]
    tnorm = jnp.sqrt(jnp.sum(t * t, axis=1, keepdims=True))
    tn = t / jnp.maximum(tnorm, 1e-12)

    sim_t = lax.dot_general(tn, protos, (((1,), (0,)), ((), ())),
                            preferred_element_type=jnp.float32) / TEMP

    valid_j = counts_col[:NUM_CLASSES, :] > 0.0
    neg_inf = jnp.float32(-jnp.inf)
    masked = jnp.where(valid_j, sim_t, neg_inf)

    m = jnp.max(masked, axis=0, keepdims=True)
    z = jnp.exp(masked - m)
    lse = jnp.log(jnp.sum(z, axis=0, keepdims=True)) + m
    logp = masked - lse

    jj = lax.broadcasted_iota(jnp.int32, (NUM_CLASSES, CPAD), 0)
    ii = lax.broadcasted_iota(jnp.int32, (NUM_CLASSES, CPAD), 1)
    eye = (jj == ii).astype(jnp.float32)
    contrib = jnp.where(valid_j, eye * logp, 0.0)
    loss_sum = -jnp.sum(contrib, axis=(0, 1), keepdims=True)

    n_valid = jnp.sum(valid_j.astype(jnp.float32), axis=(0, 1), keepdims=True)
    out_ref[...] = jnp.where(n_valid > 1.0, loss_sum / n_valid, 0.0)


def kernel(features, labels, text_embeddings, ignore_index):
    feat2d = features.reshape(BATCH * CH, HW)
    lab1d = labels.reshape(PIX).astype(jnp.int32)
    ign = jnp.full((L,), ignore_index, dtype=jnp.int32)

    sums, cnts = _sc_segsum(feat2d, lab1d, ign)

    loss2d = pl.pallas_call(
        _loss_body,
        out_shape=jax.ShapeDtypeStruct((1, 1), jnp.float32),
    )(sums.reshape(NW, CH, CPAD), cnts, cnts.T,
      text_embeddings.astype(jnp.float32))
    return loss2d[0, 0]


# speedup vs baseline: 5.7036x; 1.7851x over previous
import functools

import jax
import jax.numpy as jnp
from jax import lax
from jax.experimental import pallas as pl
from jax.experimental.pallas import tpu as pltpu
from jax.experimental.pallas import tpu_sc as plsc

TEMP = 0.1
NUM_CLASSES = 150
CPAD = 160
NC, NS, L = 2, 16, 16
NW = NC * NS
CH = 96
BATCH = 4
HW = 512 * 512
PIX = BATCH * HW
SLICE = PIX // NW
P = 4096
G = 8
CHUNKS = SLICE // P
BLOCKS = CH // G
STEPS = CHUNKS * BLOCKS
ACC_N = CH * CPAD
UNROLL = 4


def _sc_segsum_body(feat_hbm, lab_hbm, ign_hbm, sums_hbm, cnts_hbm,
                    lab_v, f0, f1, acc_v, cnt_v, ign_v, sem0, sem1):
    fbuf = (f0, f1)
    sem = (sem0, sem1)
    cid = lax.axis_index("c")
    sid = lax.axis_index("s")
    wid = sid * NC + cid
    b = wid // 8
    hw0 = (wid % 8) * SLICE

    pltpu.sync_copy(ign_hbm, ign_v)
    ign = ign_v[...]

    pltpu.sync_copy(lab_hbm.at[pl.ds(wid * SLICE, SLICE)], lab_v)

    zf = jnp.zeros((L,), jnp.float32)

    @plsc.parallel_loop(0, ACC_N // L, step=1, unroll=4)
    def _zacc(i):
        acc_v[pl.ds(i * L, L)] = zf

    @plsc.parallel_loop(0, CPAD // L, step=1, unroll=1)
    def _zcnt(i):
        cnt_v[pl.ds(i * L, L)] = zf

    def _copies(step, par):
        k = step // BLOCKS
        g = step % BLOCKS
        return [pltpu.make_async_copy(
            feat_hbm.at[b * CH + g * G + ci, pl.ds(hw0 + k * P, P)],
            fbuf[par][ci], sem[par]) for ci in range(G)]

    def issue(step, par):
        for c in _copies(step, par):
            c.start()

    def wait(step, par):
        for c in _copies(step, par):
            c.wait()

    issue(0, 0)
    issue(1, 1)

    def compute(step, par):
        k = step // BLOCKS
        g = step % BLOCKS

        @plsc.parallel_loop(0, P // L, step=1, unroll=UNROLL)
        def _pix(i):
            lv = lab_v[pl.ds(k * P + i * L, L)]
            lv = jnp.where(lv == ign, 0, lv)
            lvg = lv + g * (G * CPAD)
            for ci in range(G):
                fv = fbuf[par][ci][pl.ds(i * L, L)]
                plsc.addupdate_scatter(acc_v, [lvg + ci * CPAD], fv)

    def loop(kk, c):
        for par in range(2):
            step = kk * 2 + par
            wait(step, par)
            compute(step, par)

            @pl.when(step + 2 < STEPS)
            def _():
                issue(step + 2, par)
        return c
    lax.fori_loop(0, STEPS // 2, loop, 0)

    ones = jnp.ones((L,), jnp.float32)

    @plsc.parallel_loop(0, SLICE // L, step=1, unroll=UNROLL)
    def _cnt(i):
        lv = lab_v[pl.ds(i * L, L)]
        lv = jnp.where(lv == ign, 0, lv)
        plsc.addupdate_scatter(cnt_v, [lv], ones)

    pltpu.sync_copy(acc_v, sums_hbm.at[wid])
    pltpu.sync_copy(cnt_v, cnts_hbm.at[wid])


_sc_segsum = functools.partial(
    pl.kernel,
    out_type=(jax.ShapeDtypeStruct((NW, ACC_N), jnp.float32),
              jax.ShapeDtypeStruct((NW, CPAD), jnp.float32)),
    mesh=plsc.VectorSubcoreMesh(core_axis_name="c", subcore_axis_name="s"),
    compiler_params=pltpu.CompilerParams(needs_layout_passes=False,
                                         use_tc_tiling_on_sc=False),
    scratch_types=[
        pltpu.VMEM((SLICE,), jnp.int32),
        [pltpu.VMEM((P,), jnp.float32)] * G,
        [pltpu.VMEM((P,), jnp.float32)] * G,
        pltpu.VMEM((ACC_N,), jnp.float32),
        pltpu.VMEM((CPAD,), jnp.float32),
        pltpu.VMEM((L,), jnp.int32),
        pltpu.SemaphoreType.DMA,
        pltpu.SemaphoreType.DMA,
    ],
)(_sc_segsum_body)


def _loss_body(sums_ref, cnts_ref, cnts_t_ref, text_ref, out_ref):
    counts_row = jnp.sum(cnts_ref[...], axis=0, keepdims=True)
    counts_col = jnp.sum(cnts_t_ref[...], axis=1, keepdims=True)

    s3 = sums_ref[...]
    s = jnp.sum(s3, axis=0)
    protos = jnp.where(counts_row > 0.0,
                       s / jnp.maximum(counts_row, 1.0), 0.0)
    pnorm = jnp.sqrt(jnp.sum(protos * protos, axis=0, keepdims=True))
    protos = protos / jnp.maximum(pnorm, 1e-12)

    t = text_ref[...]
    tnorm = jnp.sqrt(jnp.sum(t * t, axis=1, keepdims=True))
    tn = t / jnp.maximum(tnorm, 1e-12)

    sim_t = lax.dot_general(tn, protos, (((1,), (0,)), ((), ())),
                            preferred_element_type=jnp.float32) / TEMP

    valid_j = counts_col[:NUM_CLASSES, :] > 0.0
    neg_inf = jnp.float32(-jnp.inf)
    masked = jnp.where(valid_j, sim_t, neg_inf)

    m = jnp.max(masked, axis=0, keepdims=True)
    z = jnp.exp(masked - m)
    lse = jnp.log(jnp.sum(z, axis=0, keepdims=True)) + m
    logp = masked - lse

    jj = lax.broadcasted_iota(jnp.int32, (NUM_CLASSES, CPAD), 0)
    ii = lax.broadcasted_iota(jnp.int32, (NUM_CLASSES, CPAD), 1)
    eye = (jj == ii).astype(jnp.float32)
    contrib = jnp.where(valid_j, eye * logp, 0.0)
    loss_sum = -jnp.sum(contrib, axis=(0, 1), keepdims=True)

    n_valid = jnp.sum(valid_j.astype(jnp.float32), axis=(0, 1), keepdims=True)
    out_ref[...] = jnp.where(n_valid > 1.0, loss_sum / n_valid, 0.0)


def kernel(features, labels, text_embeddings, ignore_index):
    feat2d = features.reshape(BATCH * CH, HW)
    lab1d = labels.reshape(PIX).astype(jnp.int32)
    ign = jnp.full((L,), ignore_index, dtype=jnp.int32)

    sums, cnts = _sc_segsum(feat2d, lab1d, ign)

    loss2d = pl.pallas_call(
        _loss_body,
        out_shape=jax.ShapeDtypeStruct((1, 1), jnp.float32),
    )(sums.reshape(NW, CH, CPAD), cnts, cnts.T,
      text_embeddings.astype(jnp.float32))
    return loss2d[0, 0]


# speedup vs baseline: 7.8531x; 1.3769x over previous
import functools

import jax
import jax.numpy as jnp
from jax import lax
from jax.experimental import pallas as pl
from jax.experimental.pallas import tpu as pltpu
from jax.experimental.pallas import tpu_sc as plsc

TEMP = 0.1
NUM_CLASSES = 150
CPAD = 160
NC, NS, L = 2, 16, 16
NW = NC * NS
CH = 96
BATCH = 4
HW = 512 * 512
PIX = BATCH * HW
SLICE = PIX // NW
P = 4096
G = 8
CHUNKS = SLICE // P
BLOCKS = CH // G
STEPS = CHUNKS * BLOCKS
ACC_N = CH * CPAD
UNROLL = 4


ROWS_SLICE = SLICE // 512
ROWS_P = P // 512


def _sc_segsum_body(feat_hbm, lab_hbm, ign_hbm, sums_hbm, cnts_hbm,
                    lab_v, f0, f1, acc_v, cnt_v, ign_v, sem0, sem1):
    fbuf = (f0, f1)
    sem = (sem0, sem1)
    cid = lax.axis_index("c")
    sid = lax.axis_index("s")
    wid = sid * NC + cid
    b = wid // 8
    r0 = (wid % 8) * ROWS_SLICE

    pltpu.sync_copy(ign_hbm, ign_v)
    ign = ign_v[...]

    for r in range(ROWS_SLICE):
        pltpu.sync_copy(lab_hbm.at[b, r0 + r], lab_v.at[pl.ds(r * 512, 512)])

    zf = jnp.zeros((L,), jnp.float32)

    @plsc.parallel_loop(0, ACC_N // L, step=1, unroll=4)
    def _zacc(i):
        acc_v[pl.ds(i * L, L)] = zf

    @plsc.parallel_loop(0, CPAD // L, step=1, unroll=1)
    def _zcnt(i):
        cnt_v[pl.ds(i * L, L)] = zf

    def _copies(step, par):
        k = step // BLOCKS
        g = step % BLOCKS
        cps = []
        for ci in range(G):
            for r in range(ROWS_P):
                cps.append(pltpu.make_async_copy(
                    feat_hbm.at[b, g * G + ci, r0 + k * ROWS_P + r],
                    fbuf[par][ci].at[pl.ds(r * 512, 512)], sem[par]))
        return cps

    def issue(step, par):
        for c in _copies(step, par):
            c.start()

    def wait(step, par):
        for c in _copies(step, par):
            c.wait()

    issue(0, 0)
    issue(1, 1)

    def compute(step, par):
        k = step // BLOCKS
        g = step % BLOCKS

        @plsc.parallel_loop(0, P // L, step=1, unroll=UNROLL)
        def _pix(i):
            lv = lab_v[pl.ds(k * P + i * L, L)]
            lv = jnp.where(lv == ign, 0, lv)
            lvg = lv + g * (G * CPAD)
            for ci in range(G):
                fv = fbuf[par][ci][pl.ds(i * L, L)]
                plsc.addupdate_scatter(acc_v, [lvg + ci * CPAD], fv)

    def loop(kk, c):
        for par in range(2):
            step = kk * 2 + par
            wait(step, par)
            compute(step, par)

            @pl.when(step + 2 < STEPS)
            def _():
                issue(step + 2, par)
        return c
    lax.fori_loop(0, STEPS // 2, loop, 0)

    ones = jnp.ones((L,), jnp.float32)

    @plsc.parallel_loop(0, SLICE // L, step=1, unroll=UNROLL)
    def _cnt(i):
        lv = lab_v[pl.ds(i * L, L)]
        lv = jnp.where(lv == ign, 0, lv)
        plsc.addupdate_scatter(cnt_v, [lv], ones)

    pltpu.sync_copy(acc_v, sums_hbm.at[pl.ds(wid * ACC_N, ACC_N)])
    pltpu.sync_copy(cnt_v, cnts_hbm.at[pl.ds(wid * CPAD, CPAD)])


_sc_segsum = functools.partial(
    pl.kernel,
    out_type=(jax.ShapeDtypeStruct((NW * ACC_N,), jnp.float32),
              jax.ShapeDtypeStruct((NW * CPAD,), jnp.float32)),
    mesh=plsc.VectorSubcoreMesh(core_axis_name="c", subcore_axis_name="s"),
    compiler_params=pltpu.CompilerParams(needs_layout_passes=False,
                                         use_tc_tiling_on_sc=True),
    scratch_types=[
        pltpu.VMEM((SLICE,), jnp.int32),
        [pltpu.VMEM((P,), jnp.float32)] * G,
        [pltpu.VMEM((P,), jnp.float32)] * G,
        pltpu.VMEM((ACC_N,), jnp.float32),
        pltpu.VMEM((CPAD,), jnp.float32),
        pltpu.VMEM((L,), jnp.int32),
        pltpu.SemaphoreType.DMA,
        pltpu.SemaphoreType.DMA,
    ],
)(_sc_segsum_body)


def _loss_body(sums_ref, cnts_ref, cnts_t_ref, text_ref, out_ref):
    counts_row = jnp.sum(cnts_ref[...], axis=0, keepdims=True)
    counts_col = jnp.sum(cnts_t_ref[...], axis=1, keepdims=True)

    s3 = sums_ref[...]
    s = jnp.sum(s3, axis=0)
    protos = jnp.where(counts_row > 0.0,
                       s / jnp.maximum(counts_row, 1.0), 0.0)
    pnorm = jnp.sqrt(jnp.sum(protos * protos, axis=0, keepdims=True))
    protos = protos / jnp.maximum(pnorm, 1e-12)

    t = text_ref[...]
    tnorm = jnp.sqrt(jnp.sum(t * t, axis=1, keepdims=True))
    tn = t / jnp.maximum(tnorm, 1e-12)

    sim_t = lax.dot_general(tn, protos, (((1,), (0,)), ((), ())),
                            preferred_element_type=jnp.float32) / TEMP

    valid_j = counts_col[:NUM_CLASSES, :] > 0.0
    neg_inf = jnp.float32(-jnp.inf)
    masked = jnp.where(valid_j, sim_t, neg_inf)

    m = jnp.max(masked, axis=0, keepdims=True)
    z = jnp.exp(masked - m)
    lse = jnp.log(jnp.sum(z, axis=0, keepdims=True)) + m
    logp = masked - lse

    jj = lax.broadcasted_iota(jnp.int32, (NUM_CLASSES, CPAD), 0)
    ii = lax.broadcasted_iota(jnp.int32, (NUM_CLASSES, CPAD), 1)
    eye = (jj == ii).astype(jnp.float32)
    contrib = jnp.where(valid_j, eye * logp, 0.0)
    loss_sum = -jnp.sum(contrib, axis=(0, 1), keepdims=True)

    n_valid = jnp.sum(valid_j.astype(jnp.float32), axis=(0, 1), keepdims=True)
    out_ref[...] = jnp.where(n_valid > 1.0, loss_sum / n_valid, 0.0)


def kernel(features, labels, text_embeddings, ignore_index):
    ign = jnp.full((L,), ignore_index, dtype=jnp.int32)

    sums, cnts = _sc_segsum(features, labels.astype(jnp.int32), ign)
    cnts2 = cnts.reshape(NW, CPAD)

    loss2d = pl.pallas_call(
        _loss_body,
        out_shape=jax.ShapeDtypeStruct((1, 1), jnp.float32),
    )(sums.reshape(NW, CH, CPAD), cnts2, cnts2.T,
      text_embeddings.astype(jnp.float32))
    return loss2d[0, 0]
